# Initial kernel scaffold; baseline (speedup 1.0000x reference)
import jax
import jax.numpy as jnp
from jax.experimental import pallas as pl


def kernel(D, senders, receivers, logm, W1, b1, W2, b2, W3, b3, W4, b4):
    raise NotImplementedError("write your pallas kernel here")


# speedup vs baseline: 20.6170x; 20.6170x over previous
import jax
import jax.numpy as jnp
from jax.experimental import pallas as pl

_NP = 16
_NE = _NP * (_NP - 1) // 2
_TB = 16


def _fused_kernel(dt_ref, l_ref, w1_ref, w2_ref, w3_ref, w4_ref, b_ref,
                  b4_ref, s_ref, invp_ref, out_ref):
    x = dt_ref[0:1, :]
    y = dt_ref[1:2, :]
    z = dt_ref[2:3, :]
    r = jnp.sqrt(x * x + y * y + z * z + 1e-12)
    ct = jnp.clip(z / r, -1.0, 1.0)
    th = jnp.arctan2(jnp.sqrt(jnp.maximum(1.0 - ct * ct, 0.0)), ct)
    ph = jnp.arctan2(y, x)
    lmr = l_ref[0:1, :]
    lms = l_ref[1:2, :]
    zero = jnp.zeros_like(r)
    feats = jnp.concatenate([r, th, ph, lmr, lms, zero, zero, zero], axis=0)

    h = jax.lax.dot_general(feats, w1_ref[...], (((0,), (0,)), ((), ())),
                            preferred_element_type=jnp.float32)
    h = jnp.tanh(h + b_ref[0:1, :])
    h = jnp.tanh(jnp.dot(h, w2_ref[...], preferred_element_type=jnp.float32)
                 + b_ref[1:2, :])
    h = jnp.tanh(jnp.dot(h, w3_ref[...], preferred_element_type=jnp.float32)
                 + b_ref[2:3, :])
    oT = jax.lax.dot_general(w4_ref[...], h, (((0,), (1,)), ((), ())),
                             preferred_element_type=jnp.float32)
    ro = oT[0:1, :] + b4_ref[0:1, 0:1]
    to = oT[1:2, :] + b4_ref[1:2, 0:1]
    po = oT[2:3, :] + b4_ref[2:3, 0:1]
    st = jnp.sin(to)
    fx = ro * st * jnp.cos(po)
    fy = ro * st * jnp.sin(po)
    fz = ro * jnp.cos(to)
    fT = jnp.concatenate([fx, fy, fz], axis=0)
    forces = jax.lax.dot_general(s_ref[...], fT, (((1,), (1,)), ((), ())),
                                 preferred_element_type=jnp.float32)
    out_ref[...] = forces * invp_ref[...]


def kernel(D, senders, receivers, logm, W1, b1, W2, b2, W3, b3, W4, b4):
    ntime = D.shape[0] // _NE
    nblocks = -(-ntime // _TB)
    ntime_p = nblocks * _TB
    if ntime_p != ntime:
        D = jnp.concatenate(
            [D, jnp.zeros(((ntime_p - ntime) * _NE, 3), D.dtype)], axis=0)

    lm = jnp.clip(logm, -12.0, 12.0)
    rt = _NE * _TB
    nt = _NP * _TB

    Dt = D.T

    lmr = jnp.tile(lm[receivers], _TB)[None, :]
    lms = jnp.tile(lm[senders], _TB)[None, :]
    L = jnp.concatenate([lmr, lms, jnp.zeros((6, rt), jnp.float32)], axis=0)

    M = (jax.nn.one_hot(receivers, _NP, dtype=jnp.float32)
         - jax.nn.one_hot(senders, _NP, dtype=jnp.float32)).T
    S = jnp.kron(jnp.eye(_TB, dtype=jnp.float32), M)

    invp = jnp.tile(jnp.power(10.0, -lm), _TB)[:, None]

    W1p = jnp.concatenate([W1, jnp.zeros((3, 128), jnp.float32)], axis=0)
    B = jnp.stack([b1, b2, b3], axis=0)
    B = jnp.concatenate([B, jnp.zeros((5, 128), jnp.float32)], axis=0)
    b4c = jnp.concatenate([b4[:, None], jnp.zeros((5, 1), jnp.float32)],
                          axis=0)

    out = pl.pallas_call(
        _fused_kernel,
        grid=(nblocks,),
        in_specs=[
            pl.BlockSpec((3, rt), lambda i: (0, i)),
            pl.BlockSpec((8, rt), lambda i: (0, 0)),
            pl.BlockSpec((8, 128), lambda i: (0, 0)),
            pl.BlockSpec((128, 128), lambda i: (0, 0)),
            pl.BlockSpec((128, 128), lambda i: (0, 0)),
            pl.BlockSpec((128, 3), lambda i: (0, 0)),
            pl.BlockSpec((8, 128), lambda i: (0, 0)),
            pl.BlockSpec((8, 1), lambda i: (0, 0)),
            pl.BlockSpec((nt, rt), lambda i: (0, 0)),
            pl.BlockSpec((nt, 1), lambda i: (0, 0)),
        ],
        out_specs=pl.BlockSpec((nt, 3), lambda i: (i, 0)),
        out_shape=jax.ShapeDtypeStruct((ntime_p * _NP, 3), jnp.float32),
    )(Dt, L, W1p, W2, W3, W4, B, b4c, S, invp)

    if ntime_p != ntime:
        out = out[: ntime * _NP]
    return out


# speedup vs baseline: 20.6557x; 1.0019x over previous
import jax
import jax.numpy as jnp
from jax.experimental import pallas as pl

_NP = 16
_NE = _NP * (_NP - 1) // 2
_TB = 16


def _fused_kernel(dt_ref, l_ref, w1_ref, w2_ref, w3_ref, w4_ref, b_ref,
                  b4_ref, s_ref, invp_ref, out_ref):
    x = dt_ref[0:1, :]
    y = dt_ref[1:2, :]
    z = dt_ref[2:3, :]
    r = jnp.sqrt(x * x + y * y + z * z + 1e-12)
    ct = jnp.clip(z / r, -1.0, 1.0)
    th = jnp.arctan2(jnp.sqrt(jnp.maximum(1.0 - ct * ct, 0.0)), ct)
    ph = jnp.arctan2(y, x)
    lmr = l_ref[0:1, :]
    lms = l_ref[1:2, :]
    zero = jnp.zeros_like(r)
    feats = jnp.concatenate([r, th, ph, lmr, lms, zero, zero, zero],
                            axis=0).astype(jnp.bfloat16)

    h = jax.lax.dot_general(feats, w1_ref[...], (((0,), (0,)), ((), ())),
                            preferred_element_type=jnp.float32)
    h = jnp.tanh(h + b_ref[0:1, :]).astype(jnp.bfloat16)
    h = jnp.tanh(jnp.dot(h, w2_ref[...], preferred_element_type=jnp.float32)
                 + b_ref[1:2, :]).astype(jnp.bfloat16)
    h = jnp.tanh(jnp.dot(h, w3_ref[...], preferred_element_type=jnp.float32)
                 + b_ref[2:3, :]).astype(jnp.bfloat16)
    oT = jax.lax.dot_general(w4_ref[...], h, (((0,), (1,)), ((), ())),
                             preferred_element_type=jnp.float32)
    ro = oT[0:1, :] + b4_ref[0:1, 0:1]
    to = oT[1:2, :] + b4_ref[1:2, 0:1]
    po = oT[2:3, :] + b4_ref[2:3, 0:1]
    st = jnp.sin(to)
    fx = ro * st * jnp.cos(po)
    fy = ro * st * jnp.sin(po)
    fz = ro * jnp.cos(to)
    fT = jnp.concatenate([fx, fy, fz], axis=0).astype(jnp.bfloat16)
    forces = jax.lax.dot_general(s_ref[...], fT, (((1,), (1,)), ((), ())),
                                 preferred_element_type=jnp.float32)
    out_ref[...] = forces * invp_ref[...]


def kernel(D, senders, receivers, logm, W1, b1, W2, b2, W3, b3, W4, b4):
    ntime = D.shape[0] // _NE
    nblocks = -(-ntime // _TB)
    ntime_p = nblocks * _TB
    if ntime_p != ntime:
        D = jnp.concatenate(
            [D, jnp.zeros(((ntime_p - ntime) * _NE, 3), D.dtype)], axis=0)

    lm = jnp.clip(logm, -12.0, 12.0)
    rt = _NE * _TB
    nt = _NP * _TB

    Dt = D.T

    lmr = jnp.tile(lm[receivers], _TB)[None, :]
    lms = jnp.tile(lm[senders], _TB)[None, :]
    L = jnp.concatenate([lmr, lms, jnp.zeros((6, rt), jnp.float32)], axis=0)

    M = (jax.nn.one_hot(receivers, _NP, dtype=jnp.float32)
         - jax.nn.one_hot(senders, _NP, dtype=jnp.float32)).T
    S = jnp.kron(jnp.eye(_TB, dtype=jnp.float32), M).astype(jnp.bfloat16)

    invp = jnp.tile(jnp.power(10.0, -lm), _TB)[:, None]

    W1p = jnp.concatenate([W1, jnp.zeros((3, 128), jnp.float32)],
                          axis=0).astype(jnp.bfloat16)
    W2 = W2.astype(jnp.bfloat16)
    W3 = W3.astype(jnp.bfloat16)
    W4 = W4.astype(jnp.bfloat16)
    B = jnp.stack([b1, b2, b3], axis=0)
    B = jnp.concatenate([B, jnp.zeros((5, 128), jnp.float32)], axis=0)
    b4c = jnp.concatenate([b4[:, None], jnp.zeros((5, 1), jnp.float32)],
                          axis=0)

    out = pl.pallas_call(
        _fused_kernel,
        grid=(nblocks,),
        in_specs=[
            pl.BlockSpec((3, rt), lambda i: (0, i)),
            pl.BlockSpec((8, rt), lambda i: (0, 0)),
            pl.BlockSpec((8, 128), lambda i: (0, 0)),
            pl.BlockSpec((128, 128), lambda i: (0, 0)),
            pl.BlockSpec((128, 128), lambda i: (0, 0)),
            pl.BlockSpec((128, 3), lambda i: (0, 0)),
            pl.BlockSpec((8, 128), lambda i: (0, 0)),
            pl.BlockSpec((8, 1), lambda i: (0, 0)),
            pl.BlockSpec((nt, rt), lambda i: (0, 0)),
            pl.BlockSpec((nt, 1), lambda i: (0, 0)),
        ],
        out_specs=pl.BlockSpec((nt, 3), lambda i: (i, 0)),
        out_shape=jax.ShapeDtypeStruct((ntime_p * _NP, 3), jnp.float32),
    )(Dt, L, W1p, W2, W3, W4, B, b4c, S, invp)

    if ntime_p != ntime:
        out = out[: ntime * _NP]
    return out


# speedup vs baseline: 23.6418x; 1.1446x over previous
import jax
import jax.numpy as jnp
from jax.experimental import pallas as pl

_NP = 16
_NE = _NP * (_NP - 1) // 2
_TA = 128
_TB = 64
_TS = 16


def _sph_kernel(d_ref, l_ref, out_ref):
    x = d_ref[0]
    y = d_ref[1]
    z = d_ref[2]
    r2 = x * x + y * y + z * z + 1e-12
    rinv = jax.lax.rsqrt(r2)
    r = r2 * rinv
    ct = z * rinv
    th = jnp.arctan2(jnp.sqrt(jnp.maximum(1.0 - ct * ct, 0.0)), ct)
    ph = jnp.arctan2(y, x)
    zero = jnp.zeros_like(r)
    out_ref[...] = jnp.stack(
        [r, th, ph, l_ref[0], l_ref[1], zero, zero, zero],
        axis=0).astype(jnp.bfloat16)


def _mlp_kernel(f_ref, w1_ref, w2_ref, w3_ref, w4_ref, b_ref, out_ref):
    h = jax.lax.dot_general(f_ref[...], w1_ref[...], (((0,), (0,)), ((), ())),
                            preferred_element_type=jnp.float32)
    h = jnp.tanh(h + b_ref[0:1, :]).astype(jnp.bfloat16)
    h = jnp.tanh(jnp.dot(h, w2_ref[...], preferred_element_type=jnp.float32)
                 + b_ref[1:2, :]).astype(jnp.bfloat16)
    h = jnp.tanh(jnp.dot(h, w3_ref[...], preferred_element_type=jnp.float32)
                 + b_ref[2:3, :]).astype(jnp.bfloat16)
    out_ref[...] = jax.lax.dot_general(
        w4_ref[...], h, (((0,), (1,)), ((), ())),
        preferred_element_type=jnp.float32)


def _cart_kernel(o_ref, b4_ref, out_ref):
    ro = o_ref[0] + b4_ref[0:1, 0:1]
    to = o_ref[1] + b4_ref[1:2, 0:1]
    po = o_ref[2] + b4_ref[2:3, 0:1]
    st = jnp.sin(to)
    fx = ro * st * jnp.cos(po)
    fy = ro * st * jnp.sin(po)
    fz = ro * jnp.cos(to)
    out_ref[...] = jnp.stack([fx, fy, fz], axis=0).astype(jnp.bfloat16)


def _agg_kernel(f_ref, s_ref, invp_ref, out_ref):
    rts = _NE * _TS
    nts = _NP * _TS
    for j in range(_TB // _TS):
        fj = f_ref[:, rts * j:rts * (j + 1)]
        forces = jax.lax.dot_general(s_ref[...], fj, (((1,), (1,)), ((), ())),
                                     preferred_element_type=jnp.float32)
        out_ref[nts * j:nts * (j + 1), :] = (
            forces * invp_ref[nts * j:nts * (j + 1), :])


def kernel(D, senders, receivers, logm, W1, b1, W2, b2, W3, b3, W4, b4):
    ntime = D.shape[0] // _NE
    nta = -(-ntime // _TA)
    ntime_p = nta * _TA
    if ntime_p != ntime:
        D = jnp.concatenate(
            [D, jnp.zeros(((ntime_p - ntime) * _NE, 3), D.dtype)], axis=0)
    E = ntime_p * _NE
    G = E // 128

    lm = jnp.clip(logm, -12.0, 12.0)

    Dt = D.T.reshape(3, G, 128)

    ga = _TA * _NE // 128
    lmr = jnp.tile(lm[receivers], _TA).reshape(ga, 128)
    lms = jnp.tile(lm[senders], _TA).reshape(ga, 128)
    L = jnp.stack([lmr, lms], axis=0)

    feats = pl.pallas_call(
        _sph_kernel,
        grid=(G // ga,),
        in_specs=[
            pl.BlockSpec((3, ga, 128), lambda i: (0, i, 0)),
            pl.BlockSpec((2, ga, 128), lambda i: (0, 0, 0)),
        ],
        out_specs=pl.BlockSpec((8, ga, 128), lambda i: (0, i, 0)),
        out_shape=jax.ShapeDtypeStruct((8, G, 128), jnp.bfloat16),
    )(Dt, L)

    W1p = jnp.concatenate([W1, jnp.zeros((3, 128), jnp.float32)],
                          axis=0).astype(jnp.bfloat16)
    B = jnp.stack([b1, b2, b3], axis=0)
    B = jnp.concatenate([B, jnp.zeros((5, 128), jnp.float32)], axis=0)
    rtb = _NE * _TB

    oT = pl.pallas_call(
        _mlp_kernel,
        grid=(E // rtb,),
        in_specs=[
            pl.BlockSpec((8, rtb), lambda i: (0, i)),
            pl.BlockSpec((8, 128), lambda i: (0, 0)),
            pl.BlockSpec((128, 128), lambda i: (0, 0)),
            pl.BlockSpec((128, 128), lambda i: (0, 0)),
            pl.BlockSpec((128, 3), lambda i: (0, 0)),
            pl.BlockSpec((8, 128), lambda i: (0, 0)),
        ],
        out_specs=pl.BlockSpec((3, rtb), lambda i: (0, i)),
        out_shape=jax.ShapeDtypeStruct((3, E), jnp.float32),
    )(feats.reshape(8, E), W1p, W2.astype(jnp.bfloat16),
      W3.astype(jnp.bfloat16), W4.astype(jnp.bfloat16), B)

    b4c = jnp.concatenate([b4[:, None], jnp.zeros((5, 1), jnp.float32)],
                          axis=0)

    f = pl.pallas_call(
        _cart_kernel,
        grid=(G // ga,),
        in_specs=[
            pl.BlockSpec((3, ga, 128), lambda i: (0, i, 0)),
            pl.BlockSpec((8, 1), lambda i: (0, 0)),
        ],
        out_specs=pl.BlockSpec((3, ga, 128), lambda i: (0, i, 0)),
        out_shape=jax.ShapeDtypeStruct((3, G, 128), jnp.bfloat16),
    )(oT.reshape(3, G, 128), b4c)

    M = (jax.nn.one_hot(receivers, _NP, dtype=jnp.float32)
         - jax.nn.one_hot(senders, _NP, dtype=jnp.float32)).T
    S = jnp.kron(jnp.eye(_TS, dtype=jnp.float32), M).astype(jnp.bfloat16)
    invp = jnp.tile(jnp.power(10.0, -lm), _TB)[:, None]

    ntb = _NP * _TB
    out = pl.pallas_call(
        _agg_kernel,
        grid=(ntime_p // _TB,),
        in_specs=[
            pl.BlockSpec((3, rtb), lambda i: (0, i)),
            pl.BlockSpec((_NP * _TS, _NE * _TS), lambda i: (0, 0)),
            pl.BlockSpec((ntb, 1), lambda i: (0, 0)),
        ],
        out_specs=pl.BlockSpec((ntb, 3), lambda i: (i, 0)),
        out_shape=jax.ShapeDtypeStruct((ntime_p * _NP, 3), jnp.float32),
    )(f.reshape(3, E), S, invp)

    if ntime_p != ntime:
        out = out[: ntime * _NP]
    return out


# speedup vs baseline: 25.6048x; 1.0830x over previous
import jax
import jax.numpy as jnp
from jax.experimental import pallas as pl

_NP = 16
_NE = _NP * (_NP - 1) // 2
_TA = 256
_TB = 128
_TS = 16


def _sph_kernel(d_ref, l_ref, out_ref):
    x = d_ref[0]
    y = d_ref[1]
    z = d_ref[2]
    r2 = x * x + y * y + z * z + 1e-12
    rinv = jax.lax.rsqrt(r2)
    r = r2 * rinv
    ct = z * rinv
    th = jnp.arctan2(jnp.sqrt(jnp.maximum(1.0 - ct * ct, 0.0)), ct)
    ph = jnp.arctan2(y, x)
    zero = jnp.zeros_like(r)
    out_ref[...] = jnp.stack(
        [r, th, ph, l_ref[0], l_ref[1], zero, zero, zero],
        axis=0).astype(jnp.bfloat16)


def _mlp_kernel(f_ref, w1_ref, w2_ref, w3_ref, w4_ref, b_ref, out_ref):
    h = jax.lax.dot_general(f_ref[...], w1_ref[...], (((0,), (0,)), ((), ())),
                            preferred_element_type=jnp.float32)
    h = jnp.tanh(h + b_ref[0:1, :]).astype(jnp.bfloat16)
    h = jnp.tanh(jnp.dot(h, w2_ref[...], preferred_element_type=jnp.float32)
                 + b_ref[1:2, :]).astype(jnp.bfloat16)
    h = jnp.tanh(jnp.dot(h, w3_ref[...], preferred_element_type=jnp.float32)
                 + b_ref[2:3, :]).astype(jnp.bfloat16)
    out_ref[...] = jax.lax.dot_general(
        w4_ref[...], h, (((0,), (1,)), ((), ())),
        preferred_element_type=jnp.float32)


def _cart_kernel(o_ref, b4_ref, out_ref):
    ro = o_ref[0] + b4_ref[0:1, 0:1]
    to = o_ref[1] + b4_ref[1:2, 0:1]
    po = o_ref[2] + b4_ref[2:3, 0:1]
    st = jnp.sin(to)
    fx = ro * st * jnp.cos(po)
    fy = ro * st * jnp.sin(po)
    fz = ro * jnp.cos(to)
    out_ref[...] = jnp.stack([fx, fy, fz], axis=0).astype(jnp.bfloat16)


def _agg_kernel(f_ref, s_ref, invp_ref, out_ref):
    rts = _NE * _TS
    nts = _NP * _TS
    for j in range(_TB // _TS):
        fj = f_ref[:, rts * j:rts * (j + 1)]
        forces = jax.lax.dot_general(s_ref[...], fj, (((1,), (1,)), ((), ())),
                                     preferred_element_type=jnp.float32)
        out_ref[nts * j:nts * (j + 1), :] = (
            forces * invp_ref[nts * j:nts * (j + 1), :])


def kernel(D, senders, receivers, logm, W1, b1, W2, b2, W3, b3, W4, b4):
    ntime = D.shape[0] // _NE
    nta = -(-ntime // _TA)
    ntime_p = nta * _TA
    if ntime_p != ntime:
        D = jnp.concatenate(
            [D, jnp.zeros(((ntime_p - ntime) * _NE, 3), D.dtype)], axis=0)
    E = ntime_p * _NE
    G = E // 128

    lm = jnp.clip(logm, -12.0, 12.0)

    Dt = D.T.reshape(3, G, 128)

    ga = _TA * _NE // 128
    lmr = jnp.tile(lm[receivers], _TA).reshape(ga, 128)
    lms = jnp.tile(lm[senders], _TA).reshape(ga, 128)
    L = jnp.stack([lmr, lms], axis=0)

    feats = pl.pallas_call(
        _sph_kernel,
        grid=(G // ga,),
        in_specs=[
            pl.BlockSpec((3, ga, 128), lambda i: (0, i, 0)),
            pl.BlockSpec((2, ga, 128), lambda i: (0, 0, 0)),
        ],
        out_specs=pl.BlockSpec((8, ga, 128), lambda i: (0, i, 0)),
        out_shape=jax.ShapeDtypeStruct((8, G, 128), jnp.bfloat16),
    )(Dt, L)

    W1p = jnp.concatenate([W1, jnp.zeros((3, 128), jnp.float32)],
                          axis=0).astype(jnp.bfloat16)
    B = jnp.stack([b1, b2, b3], axis=0)
    B = jnp.concatenate([B, jnp.zeros((5, 128), jnp.float32)], axis=0)
    rtb = _NE * _TB

    oT = pl.pallas_call(
        _mlp_kernel,
        grid=(E // rtb,),
        in_specs=[
            pl.BlockSpec((8, rtb), lambda i: (0, i)),
            pl.BlockSpec((8, 128), lambda i: (0, 0)),
            pl.BlockSpec((128, 128), lambda i: (0, 0)),
            pl.BlockSpec((128, 128), lambda i: (0, 0)),
            pl.BlockSpec((128, 3), lambda i: (0, 0)),
            pl.BlockSpec((8, 128), lambda i: (0, 0)),
        ],
        out_specs=pl.BlockSpec((3, rtb), lambda i: (0, i)),
        out_shape=jax.ShapeDtypeStruct((3, E), jnp.float32),
    )(feats.reshape(8, E), W1p, W2.astype(jnp.bfloat16),
      W3.astype(jnp.bfloat16), W4.astype(jnp.bfloat16), B)

    b4c = jnp.concatenate([b4[:, None], jnp.zeros((5, 1), jnp.float32)],
                          axis=0)

    f = pl.pallas_call(
        _cart_kernel,
        grid=(G // ga,),
        in_specs=[
            pl.BlockSpec((3, ga, 128), lambda i: (0, i, 0)),
            pl.BlockSpec((8, 1), lambda i: (0, 0)),
        ],
        out_specs=pl.BlockSpec((3, ga, 128), lambda i: (0, i, 0)),
        out_shape=jax.ShapeDtypeStruct((3, G, 128), jnp.bfloat16),
    )(oT.reshape(3, G, 128), b4c)

    M = (jax.nn.one_hot(receivers, _NP, dtype=jnp.float32)
         - jax.nn.one_hot(senders, _NP, dtype=jnp.float32)).T
    S = jnp.kron(jnp.eye(_TS, dtype=jnp.float32), M).astype(jnp.bfloat16)
    invp = jnp.tile(jnp.power(10.0, -lm), _TB)[:, None]

    ntb = _NP * _TB
    out = pl.pallas_call(
        _agg_kernel,
        grid=(ntime_p // _TB,),
        in_specs=[
            pl.BlockSpec((3, rtb), lambda i: (0, i)),
            pl.BlockSpec((_NP * _TS, _NE * _TS), lambda i: (0, 0)),
            pl.BlockSpec((ntb, 1), lambda i: (0, 0)),
        ],
        out_specs=pl.BlockSpec((ntb, 3), lambda i: (i, 0)),
        out_shape=jax.ShapeDtypeStruct((ntime_p * _NP, 3), jnp.float32),
    )(f.reshape(3, E), S, invp)

    if ntime_p != ntime:
        out = out[: ntime * _NP]
    return out


# speedup vs baseline: 36.3278x; 1.4188x over previous
import jax
import jax.numpy as jnp
from jax.experimental import pallas as pl

_NP = 16
_NE = _NP * (_NP - 1) // 2
_TA = 256
_TB = 128
_TS = 16


def _sph_kernel(d_ref, l_ref, out_ref):
    x = d_ref[0]
    y = d_ref[1]
    z = d_ref[2]
    r2 = x * x + y * y + z * z + 1e-12
    rinv = jax.lax.rsqrt(r2)
    r = r2 * rinv
    ct = z * rinv
    th = jnp.arctan2(jnp.sqrt(jnp.maximum(1.0 - ct * ct, 0.0)), ct)
    ph = jnp.arctan2(y, x)
    zero = jnp.zeros_like(r)
    out_ref[...] = jnp.stack(
        [r, th, ph, l_ref[0], l_ref[1], zero, zero, zero],
        axis=0).astype(jnp.bfloat16)


def _mlp_kernel(f_ref, w1_ref, w2_ref, w3_ref, w4_ref, b_ref, out_ref):
    h = jax.lax.dot_general(f_ref[...], w1_ref[...], (((0,), (0,)), ((), ())),
                            preferred_element_type=jnp.float32)
    h = jnp.tanh(h + b_ref[0:1, :]).astype(jnp.bfloat16)
    h = jnp.tanh(jnp.dot(h, w2_ref[...], preferred_element_type=jnp.float32)
                 + b_ref[1:2, :]).astype(jnp.bfloat16)
    h = jnp.tanh(jnp.dot(h, w3_ref[...], preferred_element_type=jnp.float32)
                 + b_ref[2:3, :]).astype(jnp.bfloat16)
    out_ref[...] = jax.lax.dot_general(
        w4_ref[...], h, (((0,), (1,)), ((), ())),
        preferred_element_type=jnp.float32)


def _cart_kernel(o_ref, b4_ref, out_ref):
    ro = o_ref[0] + b4_ref[0:1, 0:1]
    to = o_ref[1] + b4_ref[1:2, 0:1]
    po = o_ref[2] + b4_ref[2:3, 0:1]
    st = jnp.sin(to)
    fx = ro * st * jnp.cos(po)
    fy = ro * st * jnp.sin(po)
    fz = ro * jnp.cos(to)
    out_ref[...] = jnp.stack([fx, fy, fz], axis=0).astype(jnp.bfloat16)


def _agg_kernel(f_ref, mt_ref, invp_ref, out_ref):
    td = f_ref.shape[1]
    fl = f_ref[...].reshape(3 * td, _NE)
    ft = jnp.dot(fl, mt_ref[...], preferred_element_type=jnp.float32)
    ft = ft * invp_ref[...]
    out_ref[...] = ft.reshape(3, td, _NP)


def kernel(D, senders, receivers, logm, W1, b1, W2, b2, W3, b3, W4, b4):
    ntime = D.shape[0] // _NE
    nta = -(-ntime // _TA)
    ntime_p = nta * _TA
    if ntime_p != ntime:
        D = jnp.concatenate(
            [D, jnp.zeros(((ntime_p - ntime) * _NE, 3), D.dtype)], axis=0)
    E = ntime_p * _NE
    G = E // 128

    lm = jnp.clip(logm, -12.0, 12.0)

    Dt = D.T.reshape(3, G, 128)

    ga = _TA * _NE // 128
    lmr = jnp.tile(lm[receivers], _TA).reshape(ga, 128)
    lms = jnp.tile(lm[senders], _TA).reshape(ga, 128)
    L = jnp.stack([lmr, lms], axis=0)

    feats = pl.pallas_call(
        _sph_kernel,
        grid=(G // ga,),
        in_specs=[
            pl.BlockSpec((3, ga, 128), lambda i: (0, i, 0)),
            pl.BlockSpec((2, ga, 128), lambda i: (0, 0, 0)),
        ],
        out_specs=pl.BlockSpec((8, ga, 128), lambda i: (0, i, 0)),
        out_shape=jax.ShapeDtypeStruct((8, G, 128), jnp.bfloat16),
    )(Dt, L)

    W1p = jnp.concatenate([W1, jnp.zeros((3, 128), jnp.float32)],
                          axis=0).astype(jnp.bfloat16)
    B = jnp.stack([b1, b2, b3], axis=0)
    B = jnp.concatenate([B, jnp.zeros((5, 128), jnp.float32)], axis=0)
    rtb = _NE * _TB

    oT = pl.pallas_call(
        _mlp_kernel,
        grid=(E // rtb,),
        in_specs=[
            pl.BlockSpec((8, rtb), lambda i: (0, i)),
            pl.BlockSpec((8, 128), lambda i: (0, 0)),
            pl.BlockSpec((128, 128), lambda i: (0, 0)),
            pl.BlockSpec((128, 128), lambda i: (0, 0)),
            pl.BlockSpec((128, 3), lambda i: (0, 0)),
            pl.BlockSpec((8, 128), lambda i: (0, 0)),
        ],
        out_specs=pl.BlockSpec((3, rtb), lambda i: (0, i)),
        out_shape=jax.ShapeDtypeStruct((3, E), jnp.float32),
    )(feats.reshape(8, E), W1p, W2.astype(jnp.bfloat16),
      W3.astype(jnp.bfloat16), W4.astype(jnp.bfloat16), B)

    b4c = jnp.concatenate([b4[:, None], jnp.zeros((5, 1), jnp.float32)],
                          axis=0)

    f = pl.pallas_call(
        _cart_kernel,
        grid=(G // ga,),
        in_specs=[
            pl.BlockSpec((3, ga, 128), lambda i: (0, i, 0)),
            pl.BlockSpec((8, 1), lambda i: (0, 0)),
        ],
        out_specs=pl.BlockSpec((3, ga, 128), lambda i: (0, i, 0)),
        out_shape=jax.ShapeDtypeStruct((3, G, 128), jnp.bfloat16),
    )(oT.reshape(3, G, 128), b4c)

    Mt = (jax.nn.one_hot(receivers, _NP, dtype=jnp.float32)
          - jax.nn.one_hot(senders, _NP, dtype=jnp.float32)
          ).astype(jnp.bfloat16)
    invp = jnp.power(10.0, -lm)[None, :]

    td = 1024
    outT = pl.pallas_call(
        _agg_kernel,
        grid=(ntime_p // td,),
        in_specs=[
            pl.BlockSpec((3, td, _NE), lambda i: (0, i, 0)),
            pl.BlockSpec((_NE, _NP), lambda i: (0, 0)),
            pl.BlockSpec((1, _NP), lambda i: (0, 0)),
        ],
        out_specs=pl.BlockSpec((3, td, _NP), lambda i: (0, i, 0)),
        out_shape=jax.ShapeDtypeStruct((3, ntime_p, _NP), jnp.float32),
    )(f.reshape(3, ntime_p, _NE), Mt, invp)

    out = outT.reshape(3, ntime_p * _NP).T
    if ntime_p != ntime:
        out = out[: ntime * _NP]
    return out
